# baseline (speedup 1.0000x reference)
import jax
import jax.numpy as jnp
from jax import lax
from jax.experimental import pallas as pl
from jax.experimental.pallas import tpu as pltpu
from jax.experimental.pallas import tpu_sc as plsc

_NC, _NS, _L = 2, 16, 16
_NW = _NC * _NS
_CHUNK = 128
_RBLK = 2000


def _sqrt16(s):
    i = plsc.bitcast(s, jnp.int32)
    i = jnp.int32(0x5F3759DF) - lax.shift_right_logical(i, jnp.int32(1))
    y = plsc.bitcast(i, jnp.float32)
    h = jnp.float32(0.5) * s
    for _ in range(3):
        y = y * (jnp.float32(1.5) - h * y * y)
    return s * y


def _repack_kernel(ent_t, ent_b, rel_t, rel_b, ent_out, rel_out):
    ent_out[:, 0:64] = ent_t[...]
    ent_out[:, 64:128] = ent_b[...]
    rel_out[:, 0:64] = rel_t[...]
    rel_out[:, 64:128] = rel_b[...]


def _repack(ent, rel):
    nrows, dim = ent.shape
    half = nrows // 2
    grid = half // _RBLK
    hb = half // _RBLK
    return pl.pallas_call(
        _repack_kernel,
        grid=(grid,),
        in_specs=[
            pl.BlockSpec((_RBLK, dim), lambda i: (i, 0)),
            pl.BlockSpec((_RBLK, dim), lambda i, hb=hb: (i + hb, 0)),
            pl.BlockSpec((_RBLK, dim), lambda i: (i, 0)),
            pl.BlockSpec((_RBLK, dim), lambda i, hb=hb: (i + hb, 0)),
        ],
        out_specs=[
            pl.BlockSpec((_RBLK, 2 * dim), lambda i: (i, 0)),
            pl.BlockSpec((_RBLK, 2 * dim), lambda i: (i, 0)),
        ],
        out_shape=[
            jax.ShapeDtypeStruct((half, 2 * dim), jnp.float32),
            jax.ShapeDtypeStruct((half, 2 * dim), jnp.float32),
        ],
    )(ent, ent, rel, rel)


def _gather_body(per_w, dim, total):
    n_chunks = per_w // _CHUNK
    n_groups = _CHUNK // _L

    def body(idx_all, ent_p, rel_p, out,
             h_iv, r_iv, t_iv, hp_v, rp_v, tp_v,
             hrows, rrows, trows, out_v, sem0, sem1):
        wid = lax.axis_index("s") * _NC + lax.axis_index("c")
        off = wid * per_w
        iota = lax.iota(jnp.int32, _L)
        sems = (sem0, sem1)

        def load_idx(c, b):
            cbase = off + c * _CHUNK
            pltpu.sync_copy(idx_all.at[pl.ds(cbase, _CHUNK)], h_iv.at[b])
            pltpu.sync_copy(idx_all.at[pl.ds(total + cbase, _CHUNK)], r_iv.at[b])
            pltpu.sync_copy(idx_all.at[pl.ds(2 * total + cbase, _CHUNK)], t_iv.at[b])
            for k in range(n_groups):
                kb = k * _L
                for iv, pv in ((h_iv, hp_v), (r_iv, rp_v), (t_iv, tp_v)):
                    v = iv[b, pl.ds(kb, _L)]
                    hi_half = jnp.where(
                        v >= jnp.int32(500000), jnp.int32(1), jnp.int32(0))
                    pv[b, pl.ds(kb, _L)] = hi_half * jnp.int32(64)
                    iv[b, pl.ds(kb, _L)] = v - hi_half * jnp.int32(500000)

        def issue_rows(b):
            pltpu.async_copy(ent_p.at[h_iv.at[b]], hrows.at[b], sems[b])
            pltpu.async_copy(rel_p.at[r_iv.at[b]], rrows.at[b], sems[b])
            pltpu.async_copy(ent_p.at[t_iv.at[b]], trows.at[b], sems[b])

        def drain(b):
            pltpu.make_async_copy(ent_p.at[pl.ds(0, _CHUNK)], hrows.at[b], sems[b]).wait()
            pltpu.make_async_copy(ent_p.at[pl.ds(0, _CHUNK)], rrows.at[b], sems[b]).wait()
            pltpu.make_async_copy(ent_p.at[pl.ds(0, _CHUNK)], trows.at[b], sems[b]).wait()

        def compute_group(c, g, b):
            kb = g * _L
            row = kb + iota
            hp16 = hp_v[b, pl.ds(kb, _L)]
            rp16 = rp_v[b, pl.ds(kb, _L)]
            tp16 = tp_v[b, pl.ds(kb, _L)]
            accs = [jnp.zeros((_L,), jnp.float32) for _ in range(4)]
            for d in range(dim):
                col = jnp.full((_L,), d, jnp.int32)
                hv = plsc.load_gather(hrows.at[b], [row, hp16 + col])
                rv = plsc.load_gather(rrows.at[b], [row, rp16 + col])
                tv = plsc.load_gather(trows.at[b], [row, tp16 + col])
                x = hv + rv - tv
                e = jnp.exp(jnp.minimum(x, jnp.float32(20.0)))
                a = e * (e + jnp.float32(2.0))
                q = a / (a + jnp.float32(2.0))
                m = x * q
                accs[d % 4] = accs[d % 4] + m * m
            acc = (accs[0] + accs[1]) + (accs[2] + accs[3])
            out_v[pl.ds(c * _CHUNK + kb, _L)] = _sqrt16(acc)

        load_idx(0, 0)
        issue_rows(0)

        def pair_body(p, carry):
            for b in range(2):
                c = p * 2 + b

                @pl.when(c + 1 < n_chunks)
                def _():
                    load_idx(c + 1, 1 - b)
                    issue_rows(1 - b)

                drain(b)

                def g_body(g, gcarry):
                    compute_group(c, g, b)
                    return gcarry

                lax.fori_loop(0, n_groups, g_body, 0)
            return carry

        lax.fori_loop(0, n_chunks // 2, pair_body, 0)
        pltpu.sync_copy(out_v, out.at[pl.ds(off, per_w)])

    return body


@jax.jit
def _transe_pipeline(idx_all, ent, rel):
    total = idx_all.shape[0] // 3
    dim = ent.shape[1]
    ent_p, rel_p = _repack(ent, rel)

    per_w = total // _NW
    mesh = plsc.VectorSubcoreMesh(
        core_axis_name="c", subcore_axis_name="s",
        num_cores=_NC, num_subcores=_NS)
    run = pl.kernel(
        _gather_body(per_w, dim, total),
        out_type=jax.ShapeDtypeStruct((total,), jnp.float32),
        mesh=mesh,
        compiler_params=pltpu.CompilerParams(
            needs_layout_passes=False, use_tc_tiling_on_sc=True),
        scratch_types=[
            pltpu.VMEM((2, _CHUNK), jnp.int32),
            pltpu.VMEM((2, _CHUNK), jnp.int32),
            pltpu.VMEM((2, _CHUNK), jnp.int32),
            pltpu.VMEM((2, _CHUNK), jnp.int32),
            pltpu.VMEM((2, _CHUNK), jnp.int32),
            pltpu.VMEM((2, _CHUNK), jnp.int32),
            pltpu.VMEM((2, _CHUNK, 2 * dim), jnp.float32),
            pltpu.VMEM((2, _CHUNK, 2 * dim), jnp.float32),
            pltpu.VMEM((2, _CHUNK, 2 * dim), jnp.float32),
            pltpu.VMEM((per_w,), jnp.float32),
            pltpu.SemaphoreType.DMA,
            pltpu.SemaphoreType.DMA,
        ],
    )
    return run(idx_all, ent_p, rel_p)


def kernel(positive_triplets, negative_triplets, offset, entities_emb, relations_emb):
    del offset
    b = positive_triplets.shape[0]
    trip = jnp.concatenate([positive_triplets, negative_triplets], axis=0)
    idx_all = trip.T.reshape(-1)
    dist = _transe_pipeline(idx_all, entities_emb, relations_emb)
    return dist[:b], dist[b:]
